# Initial kernel scaffold; baseline (speedup 1.0000x reference)
import jax
import jax.numpy as jnp
from jax.experimental import pallas as pl


def kernel(X, mask, residue_idx, chain_labels, pe_W, pe_b, edge_W, ln_gamma, ln_beta):
    raise NotImplementedError("write your pallas kernel here")


# speedup vs baseline: 1.0351x; 1.0351x over previous
import numpy as np
import jax
import jax.numpy as jnp
from jax import lax
from jax.experimental import pallas as pl

TOP_K = 32
NUM_RBF = 16
NUM_PE = 16
MAX_REL = 32
PE_CLASSES = 2 * MAX_REL + 2
EDGE_FEATURES = 128


def _build_consts():
    sq = np.zeros((15, 75), np.float32)
    sn = np.zeros((15, 75), np.float32)
    g = np.zeros((75, 25), np.float32)
    rep = np.zeros((25, 400), np.float32)
    for a in range(5):
        for b in range(5):
            p = a * 5 + b
            for c in range(3):
                sq[3 * a + c, 3 * p + c] = 1.0
                sn[3 * b + c, 3 * p + c] = 1.0
                g[3 * p + c, p] = 1.0
            for m in range(NUM_RBF):
                rep[p, NUM_RBF * p + m] = 1.0
    mu = np.linspace(2.0, 22.0, NUM_RBF).astype(np.float32)
    mu_row = np.tile(mu, 25)[None, :]
    return sq, sn, g, rep, mu_row


_SQ, _SN, _G, _REP, _MU = _build_consts()
_INV_SIGMA = float(NUM_RBF) / 20.0


def _knn_kernel(xq_ref, cat_ref, mcol_ref, mrow_ref, resi_ref, chain_ref,
                atoms_ref, eidx_ref):
    x = xq_ref[0]
    nx, ny, nz = x[:, 0:1], x[:, 1:2], x[:, 2:3]
    cax, cay, caz = x[:, 3:4], x[:, 4:5], x[:, 5:6]
    cx, cy, cz = x[:, 6:7], x[:, 7:8], x[:, 8:9]
    ox, oy, oz = x[:, 9:10], x[:, 10:11], x[:, 11:12]
    bx, by, bz = cax - nx, cay - ny, caz - nz
    ccx, ccy, ccz = cx - cax, cy - cay, cz - caz
    ax = by * ccz - bz * ccy
    ay = bz * ccx - bx * ccz
    az = bx * ccy - by * ccx
    w1, w2, w3 = -0.58273431, 0.56802827, -0.54067466
    cbx = w1 * ax + w2 * bx + w3 * ccx + cax
    cby = w1 * ay + w2 * by + w3 * ccy + cay
    cbz = w1 * az + w2 * bz + w3 * ccz + caz
    resf = resi_ref[0].astype(jnp.float32)
    chnf = chain_ref[0].astype(jnp.float32)
    atoms_ref[0] = jnp.concatenate(
        [nx, ny, nz, cax, cay, caz, cx, cy, cz, ox, oy, oz,
         cbx, cby, cbz, resf, chnf], axis=1)

    cat = cat_ref[0]
    dx = cax - cat[0:1, :]
    dy = cay - cat[1:2, :]
    dz = caz - cat[2:3, :]
    m2 = mcol_ref[0] * mrow_ref[0]
    dist = m2 * jnp.sqrt(dx * dx + dy * dy + dz * dz + 1e-6)
    dmax = jnp.max(dist, axis=1, keepdims=True)
    work = dist + (1.0 - m2) * dmax
    iota = lax.broadcasted_iota(jnp.int32, work.shape, 1)
    big = jnp.int32(2 ** 30)
    cols = []
    for _ in range(TOP_K):
        mval = jnp.min(work, axis=1, keepdims=True)
        hit = work == mval
        sel = jnp.min(jnp.where(hit, iota, big), axis=1, keepdims=True)
        cols.append(sel)
        work = jnp.where(iota == sel, jnp.float32(jnp.inf), work)
    eidx_ref[0] = jnp.concatenate(cols, axis=1)


def _feat_kernel(idx_ref, table_ref, pew_ref, peb_ref, edgew_ref,
                 gamma_ref, beta_ref, sq_ref, sn_ref, g_ref, rep_ref, mu_ref,
                 out_ref):
    rb = idx_ref.shape[1]
    length = table_ref.shape[1]
    qpb = rb // TOP_K
    nb = pl.program_id(1)
    table = table_ref[0]
    idx = idx_ref[0]

    lane = lax.broadcasted_iota(jnp.int32, (rb, length), 1)
    oh = (lane == idx).astype(jnp.float32)
    n17 = jnp.dot(oh, table, preferred_element_type=jnp.float32, precision=lax.Precision.HIGHEST)

    qtab = table_ref[0, pl.ds(nb * qpb, qpb), :]
    sub = lax.broadcasted_iota(jnp.int32, (rb, qpb), 0) // TOP_K
    lane_q = lax.broadcasted_iota(jnp.int32, (rb, qpb), 1)
    ohq = (sub == lane_q).astype(jnp.float32)
    q17 = jnp.dot(ohq, qtab, preferred_element_type=jnp.float32, precision=lax.Precision.HIGHEST)

    q15, n15 = q17[:, 0:15], n17[:, 0:15]
    qre = jnp.round(q17[:, 15:16])
    nre = jnp.round(n17[:, 15:16])
    qch = jnp.round(q17[:, 16:17])
    nch = jnp.round(n17[:, 16:17])

    qrep = jnp.dot(q15, sq_ref[...], preferred_element_type=jnp.float32, precision=lax.Precision.HIGHEST)
    nrep = jnp.dot(n15, sn_ref[...], preferred_element_type=jnp.float32, precision=lax.Precision.HIGHEST)
    df = qrep - nrep
    s2 = jnp.dot(df * df, g_ref[...], preferred_element_type=jnp.float32, precision=lax.Precision.HIGHEST)
    d = jnp.sqrt(s2 + 1e-6)
    drep = jnp.dot(d, rep_ref[...], preferred_element_type=jnp.float32, precision=lax.Precision.HIGHEST)
    z = (drep - mu_ref[...]) * _INV_SIGMA
    rbf = jnp.exp(-(z * z))

    e_same = (jnp.abs(qch - nch) < 0.5).astype(jnp.float32)
    off = qre - nre
    dval = (jnp.clip(off + MAX_REL, 0.0, 2.0 * MAX_REL) * e_same
            + (1.0 - e_same) * (2.0 * MAX_REL + 1.0))
    lane_pe = lax.broadcasted_iota(jnp.int32, (rb, PE_CLASSES), 1)
    ohpe = (lane_pe.astype(jnp.float32) == dval).astype(jnp.float32)
    pe = jnp.dot(ohpe, pew_ref[...], preferred_element_type=jnp.float32, precision=lax.Precision.HIGHEST)
    pe = pe + peb_ref[...]

    w = edgew_ref[...]
    e = (jnp.dot(pe, w[0:NUM_PE, :], preferred_element_type=jnp.float32, precision=lax.Precision.HIGHEST)
         + jnp.dot(rbf, w[NUM_PE:, :], preferred_element_type=jnp.float32, precision=lax.Precision.HIGHEST))
    mu_ln = jnp.mean(e, axis=1, keepdims=True)
    ce = e - mu_ln
    var = jnp.mean(ce * ce, axis=1, keepdims=True)
    out_ref[0] = (ce / jnp.sqrt(var + 1e-5)) * gamma_ref[...] + beta_ref[...]


def kernel(X, mask, residue_idx, chain_labels, pe_W, pe_b, edge_W,
           ln_gamma, ln_beta):
    B, L = X.shape[0], X.shape[1]
    K = TOP_K
    x12 = X.reshape(B, L, 12)
    cat = jnp.transpose(X[:, :, 1, :], (0, 2, 1))
    mcol = mask.reshape(B, L, 1)
    mrow = mask.reshape(B, 1, L)
    resi = residue_idx.reshape(B, L, 1)
    chn = chain_labels.reshape(B, L, 1)

    RQ = 256
    atoms17, e_idx = pl.pallas_call(
        _knn_kernel,
        grid=(B, L // RQ),
        in_specs=[
            pl.BlockSpec((1, RQ, 12), lambda b, r: (b, r, 0)),
            pl.BlockSpec((1, 3, L), lambda b, r: (b, 0, 0)),
            pl.BlockSpec((1, RQ, 1), lambda b, r: (b, r, 0)),
            pl.BlockSpec((1, 1, L), lambda b, r: (b, 0, 0)),
            pl.BlockSpec((1, RQ, 1), lambda b, r: (b, r, 0)),
            pl.BlockSpec((1, RQ, 1), lambda b, r: (b, r, 0)),
        ],
        out_specs=[
            pl.BlockSpec((1, RQ, 17), lambda b, r: (b, r, 0)),
            pl.BlockSpec((1, RQ, K), lambda b, r: (b, r, 0)),
        ],
        out_shape=[
            jax.ShapeDtypeStruct((B, L, 17), jnp.float32),
            jax.ShapeDtypeStruct((B, L, K), jnp.int32),
        ],
    )(x12, cat, mcol, mrow, resi, chn)

    RB = 512
    idx_flat = e_idx.reshape(B, L * K, 1)
    E = pl.pallas_call(
        _feat_kernel,
        grid=(B, (L * K) // RB),
        in_specs=[
            pl.BlockSpec((1, RB, 1), lambda b, r: (b, r, 0)),
            pl.BlockSpec((1, L, 17), lambda b, r: (b, 0, 0)),
            pl.BlockSpec((PE_CLASSES, NUM_PE), lambda b, r: (0, 0)),
            pl.BlockSpec((1, NUM_PE), lambda b, r: (0, 0)),
            pl.BlockSpec((NUM_PE + 400, EDGE_FEATURES), lambda b, r: (0, 0)),
            pl.BlockSpec((1, EDGE_FEATURES), lambda b, r: (0, 0)),
            pl.BlockSpec((1, EDGE_FEATURES), lambda b, r: (0, 0)),
            pl.BlockSpec((15, 75), lambda b, r: (0, 0)),
            pl.BlockSpec((15, 75), lambda b, r: (0, 0)),
            pl.BlockSpec((75, 25), lambda b, r: (0, 0)),
            pl.BlockSpec((25, 400), lambda b, r: (0, 0)),
            pl.BlockSpec((1, 400), lambda b, r: (0, 0)),
        ],
        out_specs=pl.BlockSpec((1, RB, EDGE_FEATURES), lambda b, r: (b, r, 0)),
        out_shape=jax.ShapeDtypeStruct((B, L * K, EDGE_FEATURES), jnp.float32),
    )(idx_flat, atoms17, pe_W, pe_b.reshape(1, NUM_PE), edge_W,
      ln_gamma.reshape(1, EDGE_FEATURES), ln_beta.reshape(1, EDGE_FEATURES),
      jnp.asarray(_SQ), jnp.asarray(_SN), jnp.asarray(_G), jnp.asarray(_REP),
      jnp.asarray(_MU))
    return E.reshape(B, L, K, EDGE_FEATURES), e_idx


# speedup vs baseline: 1.4872x; 1.4367x over previous
import functools

import numpy as np
import jax
import jax.numpy as jnp
from jax import lax
from jax.experimental import pallas as pl
from jax.experimental.pallas import tpu as pltpu
from jax.experimental.pallas import tpu_sc as plsc

TOP_K = 32
NUM_RBF = 16
NUM_PE = 16
MAX_REL = 32
PE_CLASSES = 2 * MAX_REL + 2
EDGE_FEATURES = 128
ROW = 32

_PREC = lax.Precision.HIGHEST


def _build_consts():
    sq = np.zeros((15, 75), np.float32)
    sn = np.zeros((15, 75), np.float32)
    g = np.zeros((75, 25), np.float32)
    rep = np.zeros((25, 400), np.float32)
    for a in range(5):
        for b in range(5):
            p = a * 5 + b
            for c in range(3):
                sq[3 * a + c, 3 * p + c] = 1.0
                sn[3 * b + c, 3 * p + c] = 1.0
                g[3 * p + c, p] = 1.0
            for m in range(NUM_RBF):
                rep[p, NUM_RBF * p + m] = 1.0
    mu = np.linspace(2.0, 22.0, NUM_RBF).astype(np.float32)
    mu_row = np.tile(mu, 25)[None, :]
    return sq, sn, g, rep, mu_row


_SQ, _SN, _G, _REP, _MU = _build_consts()
_INV_SIGMA = float(NUM_RBF) / 20.0


def _knn_kernel(xq_ref, cat_ref, mcol_ref, mrow_ref, resi_ref, chain_ref,
                atoms_ref, eidx_ref, nidx_ref):
    x = xq_ref[0]
    nx, ny, nz = x[:, 0:1], x[:, 1:2], x[:, 2:3]
    cax, cay, caz = x[:, 3:4], x[:, 4:5], x[:, 5:6]
    cx, cy, cz = x[:, 6:7], x[:, 7:8], x[:, 8:9]
    ox, oy, oz = x[:, 9:10], x[:, 10:11], x[:, 11:12]
    bx, by, bz = cax - nx, cay - ny, caz - nz
    ccx, ccy, ccz = cx - cax, cy - cay, cz - caz
    ax = by * ccz - bz * ccy
    ay = bz * ccx - bx * ccz
    az = bx * ccy - by * ccx
    w1, w2, w3 = -0.58273431, 0.56802827, -0.54067466
    cbx = w1 * ax + w2 * bx + w3 * ccx + cax
    cby = w1 * ay + w2 * by + w3 * ccy + cay
    cbz = w1 * az + w2 * bz + w3 * ccz + caz
    resf = resi_ref[0].astype(jnp.float32)
    chnf = chain_ref[0].astype(jnp.float32)
    pad = jnp.zeros((x.shape[0], ROW - 17), jnp.float32)
    atoms_ref[0] = jnp.concatenate(
        [nx, ny, nz, cax, cay, caz, cx, cy, cz, ox, oy, oz,
         cbx, cby, cbz, resf, chnf, pad], axis=1)

    cat = cat_ref[0]
    dx = cax - cat[0:1, :]
    dy = cay - cat[1:2, :]
    dz = caz - cat[2:3, :]
    m2 = mcol_ref[0] * mrow_ref[0]
    dist = m2 * jnp.sqrt(dx * dx + dy * dy + dz * dz + 1e-6)
    dmax = jnp.max(dist, axis=1, keepdims=True)
    work = dist + (1.0 - m2) * dmax
    iota = lax.broadcasted_iota(jnp.int32, work.shape, 1)
    big = jnp.int32(2 ** 30)
    cols = []
    for _ in range(TOP_K):
        mval = jnp.min(work, axis=1, keepdims=True)
        hit = work == mval
        sel = jnp.min(jnp.where(hit, iota, big), axis=1, keepdims=True)
        cols.append(sel)
        work = jnp.where(iota == sel, jnp.float32(jnp.inf), work)
    eidx = jnp.concatenate(cols, axis=1)
    eidx_ref[0] = eidx
    nidx_ref[0] = eidx + pl.program_id(0) * cat.shape[1]


def _feat_kernel(ng_ref, table_ref, pew_ref, peb_ref, edgew_ref,
                 gamma_ref, beta_ref, sq_ref, sn_ref, g_ref, rep_ref, mu_ref,
                 out_ref):
    rb = ng_ref.shape[1]
    qpb = rb // TOP_K
    nb = pl.program_id(1)
    ng = ng_ref[0]

    qtab = table_ref[0, pl.ds(nb * qpb, qpb), :]
    sub = lax.broadcasted_iota(jnp.int32, (rb, qpb), 0) // TOP_K
    lane_q = lax.broadcasted_iota(jnp.int32, (rb, qpb), 1)
    ohq = (sub == lane_q).astype(jnp.float32)
    q17 = jnp.dot(ohq, qtab, preferred_element_type=jnp.float32,
                  precision=_PREC)

    q15, n15 = q17[:, 0:15], ng[:, 0:15]
    qre = jnp.round(q17[:, 15:16])
    nre = ng[:, 15:16]
    qch = jnp.round(q17[:, 16:17])
    nch = ng[:, 16:17]

    qrep = jnp.dot(q15, sq_ref[...], preferred_element_type=jnp.float32,
                   precision=_PREC)
    nrep = jnp.dot(n15, sn_ref[...], preferred_element_type=jnp.float32,
                   precision=_PREC)
    df = qrep - nrep
    s2 = jnp.dot(df * df, g_ref[...], preferred_element_type=jnp.float32,
                 precision=_PREC)
    d = jnp.sqrt(s2 + 1e-6)
    drep = jnp.dot(d, rep_ref[...], preferred_element_type=jnp.float32,
                   precision=_PREC)
    z = (drep - mu_ref[...]) * _INV_SIGMA
    rbf = jnp.exp(-(z * z))

    e_same = (jnp.abs(qch - nch) < 0.5).astype(jnp.float32)
    off = qre - nre
    dval = (jnp.clip(off + MAX_REL, 0.0, 2.0 * MAX_REL) * e_same
            + (1.0 - e_same) * (2.0 * MAX_REL + 1.0))
    lane_pe = lax.broadcasted_iota(jnp.int32, (rb, PE_CLASSES), 1)
    ohpe = (lane_pe.astype(jnp.float32) == dval).astype(jnp.float32)
    pe = jnp.dot(ohpe, pew_ref[...], preferred_element_type=jnp.float32,
                 precision=_PREC)
    pe = pe + peb_ref[...]

    w = edgew_ref[...]
    e = (jnp.dot(pe, w[0:NUM_PE, :], preferred_element_type=jnp.float32,
                 precision=_PREC)
         + jnp.dot(rbf, w[NUM_PE:, :], preferred_element_type=jnp.float32,
                   precision=_PREC))
    mu_ln = jnp.mean(e, axis=1, keepdims=True)
    ce = e - mu_ln
    var = jnp.mean(ce * ce, axis=1, keepdims=True)
    out_ref[0] = (ce / jnp.sqrt(var + 1e-5)) * gamma_ref[...] + beta_ref[...]


def _sc_gather(table, nflat):
    total = nflat.shape[0]
    info = plsc.get_sparse_core_info()
    nc, ns = info.num_cores, info.num_subcores
    nw = nc * ns
    bpw = total // nw
    chunk = 128
    nchunks = bpw // chunk
    mesh = plsc.VectorSubcoreMesh(core_axis_name="c", subcore_axis_name="s")

    @functools.partial(
        pl.kernel,
        mesh=mesh,
        compiler_params=pltpu.CompilerParams(use_tc_tiling_on_sc=False),
        out_type=jax.ShapeDtypeStruct((total, ROW), jnp.float32),
        scratch_types=[
            pltpu.VMEM((bpw,), jnp.int32),
            pltpu.VMEM((bpw, ROW), jnp.float32),
            pltpu.SemaphoreType.DMA,
        ],
    )
    def gath(table_hbm, idx_hbm, out_hbm, idx_v, rows_v, sem):
        wid = lax.axis_index("s") * nc + lax.axis_index("c")
        base = wid * bpw
        pltpu.sync_copy(idx_hbm.at[pl.ds(base, bpw)], idx_v)
        descs = []
        for j in range(nchunks):
            descs.append(pltpu.async_copy(
                table_hbm.at[idx_v.at[pl.ds(j * chunk, chunk)]],
                rows_v.at[pl.ds(j * chunk, chunk), :], sem))
        for dsc in descs:
            dsc.wait()
        pltpu.sync_copy(rows_v, out_hbm.at[pl.ds(base, bpw)])

    return gath(table, nflat)


def kernel(X, mask, residue_idx, chain_labels, pe_W, pe_b, edge_W,
           ln_gamma, ln_beta):
    B, L = X.shape[0], X.shape[1]
    K = TOP_K
    x12 = X.reshape(B, L, 12)
    cat = jnp.transpose(X[:, :, 1, :], (0, 2, 1))
    mcol = mask.reshape(B, L, 1)
    mrow = mask.reshape(B, 1, L)
    resi = residue_idx.reshape(B, L, 1)
    chn = chain_labels.reshape(B, L, 1)

    RQ = 256
    atoms, e_idx, nidx = pl.pallas_call(
        _knn_kernel,
        grid=(B, L // RQ),
        in_specs=[
            pl.BlockSpec((1, RQ, 12), lambda b, r: (b, r, 0)),
            pl.BlockSpec((1, 3, L), lambda b, r: (b, 0, 0)),
            pl.BlockSpec((1, RQ, 1), lambda b, r: (b, r, 0)),
            pl.BlockSpec((1, 1, L), lambda b, r: (b, 0, 0)),
            pl.BlockSpec((1, RQ, 1), lambda b, r: (b, r, 0)),
            pl.BlockSpec((1, RQ, 1), lambda b, r: (b, r, 0)),
        ],
        out_specs=[
            pl.BlockSpec((1, RQ, ROW), lambda b, r: (b, r, 0)),
            pl.BlockSpec((1, RQ, K), lambda b, r: (b, r, 0)),
            pl.BlockSpec((1, RQ, K), lambda b, r: (b, r, 0)),
        ],
        out_shape=[
            jax.ShapeDtypeStruct((B, L, ROW), jnp.float32),
            jax.ShapeDtypeStruct((B, L, K), jnp.int32),
            jax.ShapeDtypeStruct((B, L, K), jnp.int32),
        ],
    )(x12, cat, mcol, mrow, resi, chn)

    ngath = _sc_gather(atoms.reshape(B * L, ROW), nidx.reshape(B * L * K))
    ng = ngath.reshape(B, L * K, ROW)

    RB = 1024
    E = pl.pallas_call(
        _feat_kernel,
        grid=(B, (L * K) // RB),
        in_specs=[
            pl.BlockSpec((1, RB, ROW), lambda b, r: (b, r, 0)),
            pl.BlockSpec((1, L, ROW), lambda b, r: (b, 0, 0)),
            pl.BlockSpec((PE_CLASSES, NUM_PE), lambda b, r: (0, 0)),
            pl.BlockSpec((1, NUM_PE), lambda b, r: (0, 0)),
            pl.BlockSpec((NUM_PE + 400, EDGE_FEATURES), lambda b, r: (0, 0)),
            pl.BlockSpec((1, EDGE_FEATURES), lambda b, r: (0, 0)),
            pl.BlockSpec((1, EDGE_FEATURES), lambda b, r: (0, 0)),
            pl.BlockSpec((15, 75), lambda b, r: (0, 0)),
            pl.BlockSpec((15, 75), lambda b, r: (0, 0)),
            pl.BlockSpec((75, 25), lambda b, r: (0, 0)),
            pl.BlockSpec((25, 400), lambda b, r: (0, 0)),
            pl.BlockSpec((1, 400), lambda b, r: (0, 0)),
        ],
        out_specs=pl.BlockSpec((1, RB, EDGE_FEATURES), lambda b, r: (b, r, 0)),
        out_shape=jax.ShapeDtypeStruct((B, L * K, EDGE_FEATURES), jnp.float32),
    )(ng, atoms, pe_W, pe_b.reshape(1, NUM_PE), edge_W,
      ln_gamma.reshape(1, EDGE_FEATURES), ln_beta.reshape(1, EDGE_FEATURES),
      jnp.asarray(_SQ), jnp.asarray(_SN), jnp.asarray(_G), jnp.asarray(_REP),
      jnp.asarray(_MU))
    return E.reshape(B, L, K, EDGE_FEATURES), e_idx
